# baseline (speedup 1.0000x reference)
import functools

import jax
import jax.numpy as jnp
from jax.experimental import pallas as pl


def _route_row(xh, w, bias, eiota):
    logits = jnp.dot(w, xh, preferred_element_type=jnp.float32) + bias
    E = logits.shape[0]
    m1 = jnp.max(logits, axis=0, keepdims=True)
    i1 = jnp.min(jnp.where(logits == m1, eiota, E), axis=0, keepdims=True)
    masked = jnp.where(eiota == i1, -jnp.inf, logits)
    m2 = jnp.max(masked, axis=0, keepdims=True)
    i2 = jnp.min(jnp.where(masked == m2, eiota, E), axis=0, keepdims=True)
    ex = jnp.exp(logits - m1)
    recip = 1.0 / jnp.sum(ex, axis=0, keepdims=True)
    v1 = recip
    v2 = jnp.exp(m2 - m1) * recip
    zero = jnp.zeros_like(logits)
    gates = (jnp.where(eiota == i1, v1, zero)
             + jnp.where(eiota == i2, v2, zero))
    return gates, i1, i2, v1, v2


def _routing_body(x_ref, w_ref, b_ref, gates_ref, idx_ref, val_ref):
    w = w_ref[...]
    bias = b_ref[...]
    Hb = x_ref.shape[2]
    Wd = x_ref.shape[3]
    E = w.shape[0]
    eiota = jax.lax.broadcasted_iota(jnp.int32, (E, Wd), 0)
    for h in range(Hb):
        xh = x_ref[0, :, h, :]
        gates, i1, i2, v1, v2 = _route_row(xh, w, bias, eiota)
        gates_ref[0, :, h, :] = gates
        idx_ref[0, :, h, :] = jnp.concatenate([i1, i2], axis=0)
        val_ref[0, :, h, :] = jnp.concatenate([v1, v2], axis=0)


def _pick_hb(h):
    for t in (16, 8, 4, 2):
        if h % t == 0:
            return t
    return 1


@functools.partial(jax.jit, static_argnames=())
def kernel(x, W, b):
    B, C, H, Wd = x.shape
    E = W.shape[0]
    b2 = b.reshape(E, 1)
    Hb = _pick_hb(H)
    grid = (B, H // Hb)

    gates, idx, vals = pl.pallas_call(
        _routing_body,
        grid=grid,
        in_specs=[
            pl.BlockSpec((1, C, Hb, Wd), lambda bi, hi: (bi, 0, hi, 0)),
            pl.BlockSpec((E, C), lambda bi, hi: (0, 0)),
            pl.BlockSpec((E, 1), lambda bi, hi: (0, 0)),
        ],
        out_specs=[
            pl.BlockSpec((1, E, Hb, Wd), lambda bi, hi: (bi, 0, hi, 0)),
            pl.BlockSpec((1, 2, Hb, Wd), lambda bi, hi: (bi, 0, hi, 0)),
            pl.BlockSpec((1, 2, Hb, Wd), lambda bi, hi: (bi, 0, hi, 0)),
        ],
        out_shape=[
            jax.ShapeDtypeStruct((B, E, H, Wd), jnp.float32),
            jax.ShapeDtypeStruct((B, 2, H, Wd), jnp.int32),
            jax.ShapeDtypeStruct((B, 2, H, Wd), jnp.float32),
        ],
    )(x, W, b2)

    return gates, idx, vals


# speedup vs baseline: 1.0564x; 1.0564x over previous
import functools

import jax
import jax.numpy as jnp
from jax.experimental import pallas as pl


def _route_row(xh, w, bias, eiota):
    logits = jnp.dot(w, xh, preferred_element_type=jnp.float32) + bias
    E = logits.shape[0]
    m1 = jnp.max(logits, axis=0, keepdims=True)
    i1 = jnp.min(jnp.where(logits == m1, eiota, E), axis=0, keepdims=True)
    masked = jnp.where(eiota == i1, -jnp.inf, logits)
    m2 = jnp.max(masked, axis=0, keepdims=True)
    i2 = jnp.min(jnp.where(masked == m2, eiota, E), axis=0, keepdims=True)
    ex = jnp.exp(logits - m1)
    recip = 1.0 / jnp.sum(ex, axis=0, keepdims=True)
    v1 = recip
    v2 = jnp.exp(m2 - m1) * recip
    zero = jnp.zeros_like(logits)
    gates = (jnp.where(eiota == i1, v1, zero)
             + jnp.where(eiota == i2, v2, zero))
    return gates, i1, i2, v1, v2


def _routing_body(x_ref, w_ref, b_ref, gates_ref, idx_ref, val_ref):
    w = w_ref[...]
    bias = b_ref[...]
    Hb = x_ref.shape[2]
    Wd = x_ref.shape[3]
    E = w.shape[0]
    eiota = jax.lax.broadcasted_iota(jnp.int32, (E, Wd), 0)
    xt = jnp.moveaxis(x_ref[0], 1, 0)
    for h in range(Hb):
        xh = xt[h]
        gates, i1, i2, v1, v2 = _route_row(xh, w, bias, eiota)
        gates_ref[0, :, h, :] = gates
        idx_ref[0, :, h, :] = jnp.concatenate([i1, i2], axis=0)
        val_ref[0, :, h, :] = jnp.concatenate([v1, v2], axis=0)


def _pick_hb(h):
    for t in (16, 8, 4, 2):
        if h % t == 0:
            return t
    return 1


@functools.partial(jax.jit, static_argnames=())
def kernel(x, W, b):
    B, C, H, Wd = x.shape
    E = W.shape[0]
    b2 = b.reshape(E, 1)
    Hb = _pick_hb(H)
    grid = (B, H // Hb)

    gates, idx, vals = pl.pallas_call(
        _routing_body,
        grid=grid,
        in_specs=[
            pl.BlockSpec((1, C, Hb, Wd), lambda bi, hi: (bi, 0, hi, 0)),
            pl.BlockSpec((E, C), lambda bi, hi: (0, 0)),
            pl.BlockSpec((E, 1), lambda bi, hi: (0, 0)),
        ],
        out_specs=[
            pl.BlockSpec((1, E, Hb, Wd), lambda bi, hi: (bi, 0, hi, 0)),
            pl.BlockSpec((1, 2, Hb, Wd), lambda bi, hi: (bi, 0, hi, 0)),
            pl.BlockSpec((1, 2, Hb, Wd), lambda bi, hi: (bi, 0, hi, 0)),
        ],
        out_shape=[
            jax.ShapeDtypeStruct((B, E, H, Wd), jnp.float32),
            jax.ShapeDtypeStruct((B, 2, H, Wd), jnp.int32),
            jax.ShapeDtypeStruct((B, 2, H, Wd), jnp.float32),
        ],
    )(x, W, b2)

    return gates, idx, vals
